# baseline (speedup 1.0000x reference)
import functools

import jax
import jax.numpy as jnp
from jax import lax
from jax.experimental import pallas as pl
from jax.experimental.pallas import tpu as pltpu
from jax.experimental.pallas import tpu_sc as plsc

_HI = jax.lax.Precision.HIGHEST
_NC, _NS = 2, 16
_NW = _NC * _NS


def _tables_body(x_ref, coresT2_ref, table_ref, lflat_ref, idx_ref):
    B, D = x_ref.shape
    C = table_ref.shape[0]
    K = coresT2_ref.shape[1] // D
    ET = jnp.exp(table_ref[...])
    P = jnp.exp(coresT2_ref[...])
    M = jnp.dot(ET, P, precision=_HI, preferred_element_type=jnp.float32)
    L = jnp.log(M)
    for d in range(D):
        lflat_ref[d * C:(d + 1) * C, :] = L[:, d * K:(d + 1) * K]
    iota_d = jax.lax.broadcasted_iota(jnp.int32, (B, D), 1)
    idx_ref[...] = x_ref[...] + C * iota_d


def _logz_sc_body(D, K, BPW, idx_hbm, L_hbm, z_hbm, idx_v, rows_v, out_v, sem):
    w = lax.axis_index("s") * _NC + lax.axis_index("c")
    pltpu.sync_copy(idx_hbm.at[pl.ds(w * BPW * D, BPW * D)], idx_v)
    pltpu.async_copy(L_hbm.at[idx_v], rows_v, sem).wait()

    def body(b, carry):
        for kc in range(K // 16):
            acc = rows_v[b * D, pl.ds(kc * 16, 16)]
            for d in range(1, D):
                acc = acc + rows_v[b * D + d, pl.ds(kc * 16, 16)]
            out_v[b, pl.ds(kc * 16, 16)] = acc
        return carry

    lax.fori_loop(0, BPW, body, 0)
    pltpu.sync_copy(out_v, z_hbm.at[pl.ds(w * BPW, BPW)])


def _argmax_body(z_ref, la_ref, gk_ref, ohk_ref):
    B, K = z_ref.shape
    log_w = la_ref[...] + z_ref[...]
    mw = jnp.max(log_w, axis=1, keepdims=True)
    lse = jnp.log(jnp.sum(jnp.exp(log_w - mw), axis=1, keepdims=True)) + mw
    score = (log_w - lse) + gk_ref[...]
    m = jnp.max(score, axis=1, keepdims=True)
    iota_k = jax.lax.broadcasted_iota(jnp.int32, (B, K), 1)
    idx = jnp.where(score == m, iota_k, K)
    kmin = jnp.min(idx, axis=1, keepdims=True)
    ohk_ref[...] = (iota_k == kmin).astype(jnp.float32)


def _sample_body(cores_ref, table_ref, ohk_ref, gy_ref, x3_ref, y3_ref):
    G, K, C = cores_ref.shape
    B = ohk_ref.shape[0]
    xg = x3_ref[...].reshape(G * B, 1)
    iota_c = jax.lax.broadcasted_iota(jnp.int32, (G * B, C), 1)
    ohx = (xg == iota_c).astype(jnp.float32)
    pi = jnp.dot(ohx, table_ref[...], precision=_HI,
                 preferred_element_type=jnp.float32)
    gy = gy_ref[...].reshape(G * B, C)
    for j in range(G):
        rows = jnp.dot(ohk_ref[...], cores_ref[j], precision=_HI,
                       preferred_element_type=jnp.float32)
        sel = rows + pi[j * B:(j + 1) * B] + gy[j * B:(j + 1) * B]
        m = jnp.max(sel, axis=1, keepdims=True)
        iota_b = jax.lax.broadcasted_iota(jnp.int32, (B, C), 1)
        idx = jnp.where(sel == m, iota_b, C)
        y3_ref[j] = jnp.min(idx, axis=1, keepdims=True)


def kernel(x, log_alpha, log_cp_cores, log_pi_ref_table):
    B, D = x.shape
    K = log_alpha.shape[0]
    C = log_pi_ref_table.shape[0]
    G = 8
    BPW = B // _NW

    skey = jax.random.key(42)
    k_key, y_key = jax.random.split(skey)
    g_k = jax.random.gumbel(k_key, (B, K), jnp.float32)
    y_keys = jax.random.split(y_key, D)
    g_y = jax.vmap(lambda kk: jax.random.gumbel(kk, (B, C), jnp.float32))(y_keys)

    coresT2 = jnp.transpose(log_cp_cores, (2, 0, 1)).reshape(C, D * K)
    la = log_alpha.reshape(1, K)
    x3 = x.T.reshape(D, B, 1)

    L_flat, idx = pl.pallas_call(
        _tables_body,
        out_shape=(jax.ShapeDtypeStruct((D * C, K), jnp.float32),
                   jax.ShapeDtypeStruct((B, D), jnp.int32)),
    )(x, coresT2, log_pi_ref_table)

    z = pl.kernel(
        functools.partial(_logz_sc_body, D, K, BPW),
        out_type=jax.ShapeDtypeStruct((B, K), jnp.float32),
        mesh=plsc.VectorSubcoreMesh(core_axis_name="c", subcore_axis_name="s",
                                    num_cores=_NC, num_subcores=_NS),
        scratch_types=[pltpu.VMEM((BPW * D,), jnp.int32),
                       pltpu.VMEM((BPW * D, K), jnp.float32),
                       pltpu.VMEM((BPW, K), jnp.float32),
                       pltpu.SemaphoreType.DMA],
        compiler_params=pltpu.CompilerParams(use_tc_tiling_on_sc=False),
    )(idx.reshape(B * D), L_flat)

    ohk = pl.pallas_call(
        _argmax_body,
        out_shape=jax.ShapeDtypeStruct((B, K), jnp.float32),
    )(z, la, g_k)

    y3 = pl.pallas_call(
        _sample_body,
        grid=(D // G,),
        in_specs=[
            pl.BlockSpec((G, K, C), lambda d: (d, 0, 0)),
            pl.BlockSpec((C, C), lambda d: (0, 0)),
            pl.BlockSpec((B, K), lambda d: (0, 0)),
            pl.BlockSpec((G, B, C), lambda d: (d, 0, 0)),
            pl.BlockSpec((G, B, 1), lambda d: (d, 0, 0)),
        ],
        out_specs=pl.BlockSpec((G, B, 1), lambda d: (d, 0, 0)),
        out_shape=jax.ShapeDtypeStruct((D, B, 1), jnp.int32),
    )(log_cp_cores, log_pi_ref_table, ohk, g_y, x3)

    return y3.reshape(D, B).T


# speedup vs baseline: 1.0526x; 1.0526x over previous
import functools

import jax
import jax.numpy as jnp
from jax import lax
from jax.experimental import pallas as pl
from jax.experimental.pallas import tpu as pltpu
from jax.experimental.pallas import tpu_sc as plsc

_HI = jax.lax.Precision.HIGHEST
_NC, _NS = 2, 16
_NW = _NC * _NS

_TF_C = 0x1BD11BDA
_FONE = 0x3F800000
_TINY = float(jnp.finfo(jnp.float32).tiny)
_ROTS = ((13, 15, 26, 6), (17, 29, 16, 24))


def _rotl(x, r):
    import numpy as _np
    return jax.lax.bitwise_or(jax.lax.shift_left(x, _np.int32(r)),
                              jax.lax.shift_right_logical(x, _np.int32(32 - r)))


def _tf_gumbel(k1, k2, cnt):
    import numpy as _np
    ks2 = jax.lax.bitwise_xor(jax.lax.bitwise_xor(k1, k2), _np.int32(_TF_C))
    ks = (k1, k2, ks2)
    x0 = k1 + jnp.zeros_like(cnt)
    x1 = cnt + k2
    for i in range(5):
        for r in _ROTS[i % 2]:
            x0 = x0 + x1
            x1 = _rotl(x1, r)
            x1 = jax.lax.bitwise_xor(x0, x1)
        x0 = x0 + ks[(i + 1) % 3]
        x1 = x1 + ks[(i + 2) % 3] + _np.int32(i + 1)
    bits = jax.lax.bitwise_xor(x0, x1)
    fb = jax.lax.bitwise_or(jax.lax.shift_right_logical(bits, _np.int32(9)),
                            _np.int32(_FONE))
    u = jax.lax.bitcast_convert_type(fb, jnp.float32) - _np.float32(1.0)
    tiny = _np.float32(_TINY)
    u = jnp.maximum(tiny, u * (_np.float32(1.0) - tiny) + tiny)
    return -jnp.log(-jnp.log(u))


def _tables_body(x_ref, coresT2_ref, table_ref, lflat_ref, idx_ref):
    B, D = x_ref.shape
    C = table_ref.shape[0]
    K = coresT2_ref.shape[1] // D
    ET = jnp.exp(table_ref[...])
    P = jnp.exp(coresT2_ref[...])
    M = jnp.dot(ET, P, precision=_HI, preferred_element_type=jnp.float32)
    L = jnp.log(M)
    for d in range(D):
        lflat_ref[d * C:(d + 1) * C, :] = L[:, d * K:(d + 1) * K]
    iota_d = jax.lax.broadcasted_iota(jnp.int32, (B, D), 1)
    idx_ref[...] = x_ref[...] + C * iota_d


def _logz_sc_body(D, K, BPW, idx_hbm, L_hbm, z_hbm, idx_v, rows_v, out_v, sem):
    w = lax.axis_index("s") * _NC + lax.axis_index("c")
    pltpu.sync_copy(idx_hbm.at[pl.ds(w * BPW * D, BPW * D)], idx_v)
    pltpu.async_copy(L_hbm.at[idx_v], rows_v, sem).wait()

    def body(b, carry):
        for kc in range(K // 16):
            acc = rows_v[b * D, pl.ds(kc * 16, 16)]
            for d in range(1, D):
                acc = acc + rows_v[b * D + d, pl.ds(kc * 16, 16)]
            out_v[b, pl.ds(kc * 16, 16)] = acc
        return carry

    lax.fori_loop(0, BPW, body, 0)
    pltpu.sync_copy(out_v, z_hbm.at[pl.ds(w * BPW, BPW)])


def _argmax_body(z_ref, la_ref, kk_ref, ohk_ref):
    B, K = z_ref.shape
    cnt = (jax.lax.broadcasted_iota(jnp.int32, (B, K), 0) * K
           + jax.lax.broadcasted_iota(jnp.int32, (B, K), 1))
    g_k = _tf_gumbel(kk_ref[0:1, 0:1], kk_ref[0:1, 1:2], cnt)
    log_w = la_ref[...] + z_ref[...]
    mw = jnp.max(log_w, axis=1, keepdims=True)
    lse = jnp.log(jnp.sum(jnp.exp(log_w - mw), axis=1, keepdims=True)) + mw
    score = (log_w - lse) + g_k
    m = jnp.max(score, axis=1, keepdims=True)
    iota_k = jax.lax.broadcasted_iota(jnp.int32, (B, K), 1)
    idx = jnp.where(score == m, iota_k, K)
    kmin = jnp.min(idx, axis=1, keepdims=True)
    ohk_ref[...] = (iota_k == kmin).astype(jnp.float32)


def _sample_body(yk_ref, cores_ref, table_ref, ohk_ref, x3_ref, y3_ref):
    G, K, C = cores_ref.shape
    B = ohk_ref.shape[0]
    xg = x3_ref[...].reshape(G * B, 1)
    iota_c = jax.lax.broadcasted_iota(jnp.int32, (G * B, C), 1)
    ohx = (xg == iota_c).astype(jnp.float32)
    pi = jnp.dot(ohx, table_ref[...], precision=_HI,
                 preferred_element_type=jnp.float32)
    cnt = (jax.lax.broadcasted_iota(jnp.int32, (B, C), 0) * C
           + jax.lax.broadcasted_iota(jnp.int32, (B, C), 1))
    for j in range(G):
        gy = _tf_gumbel(yk_ref[j, 0:1, 0:1], yk_ref[j, 0:1, 1:2], cnt)
        rows = jnp.dot(ohk_ref[...], cores_ref[j], precision=_HI,
                       preferred_element_type=jnp.float32)
        sel = rows + pi[j * B:(j + 1) * B] + gy
        m = jnp.max(sel, axis=1, keepdims=True)
        iota_b = jax.lax.broadcasted_iota(jnp.int32, (B, C), 1)
        idx = jnp.where(sel == m, iota_b, C)
        y3_ref[j] = jnp.min(idx, axis=1, keepdims=True)


def kernel(x, log_alpha, log_cp_cores, log_pi_ref_table):
    B, D = x.shape
    K = log_alpha.shape[0]
    C = log_pi_ref_table.shape[0]
    G = 8
    BPW = B // _NW

    skey = jax.random.key(42)
    k_key, y_key = jax.random.split(skey)
    y_keys = jax.random.split(y_key, D)
    kk = jax.lax.bitcast_convert_type(
        jax.random.key_data(k_key), jnp.int32).reshape(1, 2)
    ykd = jax.lax.bitcast_convert_type(
        jax.vmap(jax.random.key_data)(y_keys), jnp.int32).reshape(D, 1, 2)

    coresT2 = jnp.transpose(log_cp_cores, (2, 0, 1)).reshape(C, D * K)
    la = log_alpha.reshape(1, K)
    x3 = x.T.reshape(D, B, 1)

    L_flat, idx = pl.pallas_call(
        _tables_body,
        out_shape=(jax.ShapeDtypeStruct((D * C, K), jnp.float32),
                   jax.ShapeDtypeStruct((B, D), jnp.int32)),
    )(x, coresT2, log_pi_ref_table)

    z = pl.kernel(
        functools.partial(_logz_sc_body, D, K, BPW),
        out_type=jax.ShapeDtypeStruct((B, K), jnp.float32),
        mesh=plsc.VectorSubcoreMesh(core_axis_name="c", subcore_axis_name="s",
                                    num_cores=_NC, num_subcores=_NS),
        scratch_types=[pltpu.VMEM((BPW * D,), jnp.int32),
                       pltpu.VMEM((BPW * D, K), jnp.float32),
                       pltpu.VMEM((BPW, K), jnp.float32),
                       pltpu.SemaphoreType.DMA],
        compiler_params=pltpu.CompilerParams(use_tc_tiling_on_sc=False),
    )(idx.reshape(B * D), L_flat)

    ohk = pl.pallas_call(
        _argmax_body,
        out_shape=jax.ShapeDtypeStruct((B, K), jnp.float32),
    )(z, la, kk)

    y3 = pl.pallas_call(
        _sample_body,
        grid=(D // G,),
        in_specs=[
            pl.BlockSpec((G, 1, 2), lambda d: (d, 0, 0)),
            pl.BlockSpec((G, K, C), lambda d: (d, 0, 0)),
            pl.BlockSpec((C, C), lambda d: (0, 0)),
            pl.BlockSpec((B, K), lambda d: (0, 0)),
            pl.BlockSpec((G, B, 1), lambda d: (d, 0, 0)),
        ],
        out_specs=pl.BlockSpec((G, B, 1), lambda d: (d, 0, 0)),
        out_shape=jax.ShapeDtypeStruct((D, B, 1), jnp.int32),
    )(ykd, log_cp_cores, log_pi_ref_table, ohk, x3)

    return y3.reshape(D, B).T
